# Initial kernel scaffold; baseline (speedup 1.0000x reference)
import jax
import jax.numpy as jnp
from jax.experimental import pallas as pl


def kernel(A, R_weight, C_weight, W, report_batch, code_i_batch, code_j_batch):
    raise NotImplementedError("write your pallas kernel here")


# speedup vs baseline: 13.9218x; 13.9218x over previous
import functools

import jax
import jax.numpy as jnp
from jax import lax
from jax.experimental import pallas as pl
from jax.experimental.pallas import tpu as pltpu
from jax.experimental.pallas import tpu_sc as plsc

R_SIZE = 10000
C_SIZE = 5000
FEAT = 128
BATCH = 4096

CB = 200
NC_GRID = C_SIZE // CB

_F32 = jnp.float32


def _dot(a, b, dims):
    return lax.dot_general(a, b, (dims, ((), ())),
                           preferred_element_type=_F32)


def _embed_body(at_ref, rw_ref, cw_ref, w_ref, hr_ref, hc_ref,
                rowagg_ref, rowdeg_ref):
    i = pl.program_id(0)
    w1 = w_ref[:FEAT, :]
    w2 = w_ref[FEAT:, :]
    at = at_ref[...]
    atb = at.astype(jnp.bfloat16)
    rwb = rw_ref[...].astype(jnp.bfloat16)
    cwb = cw_ref[...].astype(jnp.bfloat16)

    coldeg = jnp.maximum(jnp.sum(at, axis=1, keepdims=True), 1.0)
    neigh_c = _dot(atb, rwb, ((1,), (0,))) / coldeg
    hcin = jnp.concatenate([cw_ref[...], neigh_c], axis=1)
    hc = _dot(hcin, w_ref[...], ((1,), (0,)))
    hc_ref[...] = jnp.maximum(hc, 0.0)

    part_t = _dot(cwb, atb, ((0,), (0,)))
    psum = jnp.sum(at, axis=0, keepdims=True)

    @pl.when(i == 0)
    def _init():
        rowagg_ref[...] = part_t
        rowdeg_ref[...] = psum

    @pl.when(i > 0)
    def _acc():
        rowagg_ref[...] += part_t
        rowdeg_ref[...] += psum

    @pl.when(i == NC_GRID - 1)
    def _finish():
        recip = 1.0 / jnp.maximum(rowdeg_ref[...], 1.0)
        ones = jnp.ones((1, FEAT), _F32)
        deg_b = _dot(recip, ones, ((0,), (0,)))
        hr = (_dot(rw_ref[...], w1, ((1,), (0,)))
              + _dot(rowagg_ref[...], w2, ((0,), (0,))) * deg_b)
        hr_ref[...] = jnp.maximum(hr, 0.0)


def _embed(A, R_weight, C_weight, W):
    return pl.pallas_call(
        _embed_body,
        grid=(NC_GRID,),
        in_specs=[
            pl.BlockSpec((CB, R_SIZE), lambda i: (i, 0)),
            pl.BlockSpec((R_SIZE, FEAT), lambda i: (0, 0)),
            pl.BlockSpec((CB, FEAT), lambda i: (i, 0)),
            pl.BlockSpec((2 * FEAT, FEAT), lambda i: (0, 0)),
        ],
        out_specs=[
            pl.BlockSpec((R_SIZE, FEAT), lambda i: (0, 0)),
            pl.BlockSpec((CB, FEAT), lambda i: (i, 0)),
        ],
        out_shape=[
            jax.ShapeDtypeStruct((R_SIZE, FEAT), _F32),
            jax.ShapeDtypeStruct((C_SIZE, FEAT), _F32),
        ],
        scratch_shapes=[
            pltpu.VMEM((FEAT, R_SIZE), _F32),
            pltpu.VMEM((1, R_SIZE), _F32),
        ],
        compiler_params=pltpu.CompilerParams(
            dimension_semantics=("arbitrary",),
        ),
    )(A.T, R_weight, C_weight, W)


_NC = 2
_NS = 16
_NW = _NC * _NS
_BPW = BATCH // _NW
_GROUPS = _BPW // 16


def _score_body(hr_hbm, hc_hbm, rb_hbm, ci_hbm, cj_hbm,
                oi_hbm, oj_hbm,
                rb_v, ci_v, cj_v, hr_v, hci_v, hcj_v, oi_v, oj_v,
                sem1, sem2, sem3):
    wid = lax.axis_index("s") * _NC + lax.axis_index("c")
    base = wid * _BPW

    pltpu.sync_copy(rb_hbm.at[pl.ds(base, _BPW)], rb_v)
    pltpu.sync_copy(ci_hbm.at[pl.ds(base, _BPW)], ci_v)
    pltpu.sync_copy(cj_hbm.at[pl.ds(base, _BPW)], cj_v)

    cp1 = pltpu.async_copy(hr_hbm.at[rb_v], hr_v, sem1)
    cp2 = pltpu.async_copy(hc_hbm.at[ci_v], hci_v, sem2)
    cp3 = pltpu.async_copy(hc_hbm.at[cj_v], hcj_v, sem3)
    cp1.wait()
    cp2.wait()
    cp3.wait()

    lane = lax.iota(jnp.int32, 16)

    def per_group(g, _):
        def per_row(r, carry):
            res_i, res_j = carry
            b = g * 16 + r
            ai = jnp.zeros((16,), _F32)
            aj = jnp.zeros((16,), _F32)
            for k in range(FEAT // 16):
                rv = hr_v[b, pl.ds(k * 16, 16)]
                ai += rv * hci_v[b, pl.ds(k * 16, 16)]
                aj += rv * hcj_v[b, pl.ds(k * 16, 16)]
            res_i = jnp.where(lane == r, jnp.sum(ai), res_i)
            res_j = jnp.where(lane == r, jnp.sum(aj), res_j)
            return res_i, res_j

        zero = jnp.zeros((16,), _F32)
        res_i, res_j = lax.fori_loop(0, 16, per_row, (zero, zero))
        oi_v[pl.ds(g * 16, 16)] = res_i
        oj_v[pl.ds(g * 16, 16)] = res_j
        return 0

    lax.fori_loop(0, _GROUPS, per_group, 0)

    pltpu.sync_copy(oi_v, oi_hbm.at[pl.ds(base, _BPW)])
    pltpu.sync_copy(oj_v, oj_hbm.at[pl.ds(base, _BPW)])


@functools.lru_cache(maxsize=1)
def _get_score():
    return functools.partial(
        pl.kernel,
        mesh=plsc.VectorSubcoreMesh(core_axis_name="c", subcore_axis_name="s"),
        compiler_params=pltpu.CompilerParams(needs_layout_passes=False),
        out_type=[
            jax.ShapeDtypeStruct((BATCH,), _F32),
            jax.ShapeDtypeStruct((BATCH,), _F32),
        ],
        scratch_types=[
            pltpu.VMEM((_BPW,), jnp.int32),
            pltpu.VMEM((_BPW,), jnp.int32),
            pltpu.VMEM((_BPW,), jnp.int32),
            pltpu.VMEM((_BPW, FEAT), _F32),
            pltpu.VMEM((_BPW, FEAT), _F32),
            pltpu.VMEM((_BPW, FEAT), _F32),
            pltpu.VMEM((_BPW,), _F32),
            pltpu.VMEM((_BPW,), _F32),
            pltpu.SemaphoreType.DMA,
            pltpu.SemaphoreType.DMA,
            pltpu.SemaphoreType.DMA,
        ],
    )(_score_body)


def kernel(A, R_weight, C_weight, W, report_batch, code_i_batch, code_j_batch):
    hr, hc = _embed(A, R_weight, C_weight, W)
    pred_i, pred_j = _get_score()(hr, hc,
                            report_batch.astype(jnp.int32),
                            code_i_batch.astype(jnp.int32),
                            code_j_batch.astype(jnp.int32))
    return (pred_i, pred_j)
